# Initial kernel scaffold; baseline (speedup 1.0000x reference)
import jax
import jax.numpy as jnp
from jax.experimental import pallas as pl


def kernel(x, edge_index, edge_weight, W):
    raise NotImplementedError("write your pallas kernel here")


# speedup vs baseline: 3.7001x; 3.7001x over previous
import dataclasses
import functools

import jax
import jax.numpy as jnp
from jax import lax
from jax.experimental import pallas as pl
from jax.experimental.pallas import tpu as pltpu
from jax.experimental.pallas import tpu_sc as plsc

N = 10000
E = 320000
D = 128

NC = 2
NS = 16
NW = NC * NS
EPW = E // NW
C = 80
NCHUNK = EPW // C
RB = 80
NRB = N // RB
LANES = 16


def _matmul_body(x_ref, wt_ref, h_ref):
    h_ref[...] = jnp.dot(x_ref[...], wt_ref[...],
                         preferred_element_type=jnp.float32)


def _combine_body(p0_ref, p1_ref, o_ref):
    o_ref[...] = p0_ref[...] + p1_ref[...]


def _sc_spmm(h, src, dst, w):
    mesh = plsc.VectorSubcoreMesh(core_axis_name="c", subcore_axis_name="s")
    cp = pltpu.CompilerParams()
    if "needs_layout_passes" in pltpu.CompilerParams.__dataclass_fields__:
        cp = dataclasses.replace(cp, needs_layout_passes=False)

    @functools.partial(
        pl.kernel,
        compiler_params=cp,
        out_type=jax.ShapeDtypeStruct((NC, N, D), jnp.float32),
        mesh=mesh,
        scratch_types=[
            pltpu.VMEM((C,), jnp.int32),
            pltpu.VMEM((C,), jnp.int32),
            pltpu.VMEM((C,), jnp.float32),
            pltpu.VMEM((C, D), jnp.float32),
            pltpu.VMEM((RB, D), jnp.float32),
            pltpu.VMEM_SHARED((N, D), jnp.float32),
            pltpu.SemaphoreType.DMA,
        ],
    )
    def spmm_kernel(h_hbm, src_hbm, dst_hbm, w_hbm, out_hbm,
                    src_v, dst_v, w_v, rows_v, zero_v, acc, sem):
        core = lax.axis_index("c")
        sub = lax.axis_index("s")
        wid = sub * NC + core

        zeros16 = jnp.zeros((LANES,), jnp.float32)

        @pl.loop(0, RB)
        def _zero_fill(r):
            for j in range(D // LANES):
                zero_v[r, pl.ds(j * LANES, LANES)] = zeros16

        @pl.loop(sub, NRB, step=NS)
        def _zero_acc(b):
            pltpu.sync_copy(zero_v, acc.at[pl.ds(b * RB, RB)])

        plsc.subcore_barrier()

        base = wid * EPW

        @pl.loop(0, NCHUNK)
        def _edge_chunk(ci):
            off = base + ci * C
            pltpu.sync_copy(src_hbm.at[pl.ds(off, C)], src_v)
            pltpu.sync_copy(dst_hbm.at[pl.ds(off, C)], dst_v)
            pltpu.sync_copy(w_hbm.at[pl.ds(off, C)], w_v)
            pltpu.async_copy(h_hbm.at[src_v], rows_v, sem).wait()

            @pl.loop(0, C, step=LANES)
            def _scale(e0):
                for k in range(LANES):
                    e = e0 + k
                    widx = jnp.full((LANES,), e, jnp.int32)
                    wsplat = plsc.load_gather(w_v, [widx])
                    for j in range(D // LANES):
                        sl = (e, pl.ds(j * LANES, LANES))
                        rows_v[sl] = rows_v[sl] * wsplat

            pltpu.sync_copy(rows_v, acc.at[dst_v], add=True)

        plsc.subcore_barrier()

        @pl.loop(sub, NRB, step=NS)
        def _write_out(b):
            pltpu.sync_copy(acc.at[pl.ds(b * RB, RB)],
                            out_hbm.at[core, pl.ds(b * RB, RB)])

    return spmm_kernel(h, src, dst, w)


def _matmul(x, wt):
    rows = 1000
    return pl.pallas_call(
        _matmul_body,
        out_shape=jax.ShapeDtypeStruct((N, D), jnp.float32),
        grid=(N // rows,),
        in_specs=[
            pl.BlockSpec((rows, D), lambda i: (i, 0)),
            pl.BlockSpec((D, D), lambda i: (0, 0)),
        ],
        out_specs=pl.BlockSpec((rows, D), lambda i: (i, 0)),
    )(x, wt)


def _combine(p0, p1):
    rows = 1000
    return pl.pallas_call(
        _combine_body,
        out_shape=jax.ShapeDtypeStruct((N, D), jnp.float32),
        grid=(N // rows,),
        in_specs=[
            pl.BlockSpec((rows, D), lambda i: (i, 0)),
            pl.BlockSpec((rows, D), lambda i: (i, 0)),
        ],
        out_specs=pl.BlockSpec((rows, D), lambda i: (i, 0)),
    )(p0, p1)


@jax.jit
def kernel(x, edge_index, edge_weight, W):
    h = _matmul(x, W.T)
    partial = _sc_spmm(h, edge_index[1], edge_index[0], edge_weight)
    return _combine(partial[0], partial[1])


# speedup vs baseline: 7.1880x; 1.9427x over previous
import dataclasses
import functools

import jax
import jax.numpy as jnp
from jax import lax
from jax.experimental import pallas as pl
from jax.experimental.pallas import tpu as pltpu
from jax.experimental.pallas import tpu_sc as plsc

N = 10000
E = 320000
D = 128

NC = 2
NS = 16
NW = NC * NS
EPW = E // NW
C = 80
NCHUNK = EPW // C
RB = 80
NRB = N // RB
LANES = 16


def _matmul_body(x_ref, wt_ref, h_ref):
    h_ref[...] = jnp.dot(x_ref[...], wt_ref[...],
                         preferred_element_type=jnp.float32)


def _combine_body(p0_ref, p1_ref, o_ref):
    o_ref[...] = p0_ref[...] + p1_ref[...]


def _sc_spmm(h, pk):
    mesh = plsc.VectorSubcoreMesh(core_axis_name="c", subcore_axis_name="s")
    cp = pltpu.CompilerParams()
    if "needs_layout_passes" in pltpu.CompilerParams.__dataclass_fields__:
        cp = dataclasses.replace(cp, needs_layout_passes=False)

    @functools.partial(
        pl.kernel,
        compiler_params=cp,
        out_type=jax.ShapeDtypeStruct((NC, N, D), jnp.float32),
        mesh=mesh,
        scratch_types=[
            pltpu.VMEM((3, C), jnp.int32),
            pltpu.VMEM((3, C), jnp.int32),
            pltpu.VMEM((C,), jnp.int32),
            pltpu.VMEM((C, D), jnp.float32),
            pltpu.VMEM((C, D), jnp.float32),
            pltpu.VMEM((C, D), jnp.float32),
            pltpu.VMEM_SHARED((N, D), jnp.float32),
            pltpu.SemaphoreType.DMA,
            pltpu.SemaphoreType.DMA,
            pltpu.SemaphoreType.DMA,
            pltpu.SemaphoreType.DMA,
            pltpu.SemaphoreType.DMA,
        ],
    )
    def spmm_kernel(h_hbm, pk_hbm, out_hbm,
                    ib0, ib1, dst_s, g0, g1, s, acc,
                    i_sem0, i_sem1, g_sem0, g_sem1, s_sem):
        core = lax.axis_index("c")
        sub = lax.axis_index("s")
        wid = sub * NC + core

        ibs = (ib0, ib1)
        isems = (i_sem0, i_sem1)
        gbufs = (g0, g1)
        gsems = (g_sem0, g_sem1)
        zi16 = jnp.zeros((LANES,), jnp.int32)
        two16 = jnp.full((LANES,), 2, jnp.int32)
        zeros16 = jnp.zeros((LANES,), jnp.float32)

        def start_idx(c, b):
            pltpu.async_copy(pk_hbm.at[wid, c], ibs[b], isems[b])

        def wait_idx(c, b):
            pltpu.make_async_copy(pk_hbm.at[wid, c], ibs[b], isems[b]).wait()

        def start_gather(b):
            pltpu.async_copy(h_hbm.at[ibs[b].at[0]], gbufs[b], gsems[b])

        def wait_gather(b):
            pltpu.make_async_copy(
                h_hbm.at[ibs[b].at[0]], gbufs[b], gsems[b]).wait()

        def start_scatter():
            pltpu.async_copy(s, acc.at[dst_s], s_sem, add=True)

        def wait_scatter():
            pltpu.make_async_copy(s, acc.at[dst_s], s_sem).wait()

        def copy_dst(b):
            for q in range(C // LANES):
                sl = pl.ds(q * LANES, LANES)
                dst_s[sl] = ibs[b][1, sl]

        def scale(c, b):
            gb = gbufs[b]

            @pl.loop(0, C, step=LANES)
            def _scale(e0):
                for k in range(LANES):
                    e = e0 + k
                    wbits = plsc.load_gather(ibs[b], [two16, zi16 + e])
                    wsplat = plsc.bitcast(wbits, jnp.float32)
                    for j in range(D // LANES):
                        sl = (e, pl.ds(j * LANES, LANES))
                        s[sl] = gb[sl] * wsplat

        start_idx(0, 0)
        start_idx(1, 1)

        @pl.loop(0, RB)
        def _zero_fill(r):
            for j in range(D // LANES):
                g0[r, pl.ds(j * LANES, LANES)] = zeros16

        @pl.loop(sub, NRB, step=NS)
        def _zero_acc(b):
            pltpu.sync_copy(g0, acc.at[pl.ds(b * RB, RB)])

        plsc.subcore_barrier()

        wait_idx(0, 0)
        start_gather(0)

        @pl.loop(0, NCHUNK - 1, step=2)
        def _edge_chunks(ci):
            for b in range(2):
                c = ci + b
                wait_idx(c + 1, 1 - b)
                start_gather(1 - b)

                wait_gather(b)

                @pl.when(c > 0)
                def _():
                    wait_scatter()

                copy_dst(b)
                scale(c, b)
                start_scatter()

                if b == 0:
                    start_idx(c + 2, b)
                else:
                    @pl.when(c + 2 < NCHUNK)
                    def _():
                        start_idx(c + 2, b)

        wait_gather(0)
        wait_scatter()
        copy_dst(0)
        scale(NCHUNK - 1, 0)
        start_scatter()
        wait_scatter()

        plsc.subcore_barrier()

        @pl.loop(sub, NRB, step=NS)
        def _write_out(b):
            pltpu.sync_copy(acc.at[pl.ds(b * RB, RB)],
                            out_hbm.at[core, pl.ds(b * RB, RB)])

    return spmm_kernel(h, pk)


def _matmul(x, wt):
    rows = 1000
    return pl.pallas_call(
        _matmul_body,
        out_shape=jax.ShapeDtypeStruct((N, D), jnp.float32),
        grid=(N // rows,),
        in_specs=[
            pl.BlockSpec((rows, D), lambda i: (i, 0)),
            pl.BlockSpec((D, D), lambda i: (0, 0)),
        ],
        out_specs=pl.BlockSpec((rows, D), lambda i: (i, 0)),
    )(x, wt)


def _combine(p0, p1):
    rows = 1000
    return pl.pallas_call(
        _combine_body,
        out_shape=jax.ShapeDtypeStruct((N, D), jnp.float32),
        grid=(N // rows,),
        in_specs=[
            pl.BlockSpec((rows, D), lambda i: (i, 0)),
            pl.BlockSpec((rows, D), lambda i: (i, 0)),
        ],
        out_specs=pl.BlockSpec((rows, D), lambda i: (i, 0)),
    )(p0, p1)


@jax.jit
def kernel(x, edge_index, edge_weight, W):
    h = _matmul(x, W.T)
    src = edge_index[1].reshape(NW, NCHUNK, C)
    dst = edge_index[0].reshape(NW, NCHUNK, C)
    wbits = lax.bitcast_convert_type(edge_weight,
                                     jnp.int32).reshape(NW, NCHUNK, C)
    pk = jnp.stack([src, dst, wbits], axis=2)
    partial = _sc_spmm(h, pk)
    return _combine(partial[0], partial[1])


# speedup vs baseline: 8.1386x; 1.1323x over previous
import dataclasses
import functools

import jax
import jax.numpy as jnp
from jax import lax
from jax.experimental import pallas as pl
from jax.experimental.pallas import tpu as pltpu
from jax.experimental.pallas import tpu_sc as plsc

N = 10000
E = 320000
D = 128

NC = 2
NS = 16
NW = NC * NS
EPW = E // NW
C = 80
NCHUNK = EPW // C
RB = 80
NRB = N // RB
LANES = 16


def _matmul_body(x_ref, wt_ref, h_ref):
    h_ref[...] = jnp.dot(x_ref[...], wt_ref[...],
                         preferred_element_type=jnp.float32)


def _combine_body(p0_ref, p1_ref, o_ref):
    o_ref[...] = p0_ref[...] + p1_ref[...]


def _sc_spmm(h, pk):
    mesh = plsc.VectorSubcoreMesh(core_axis_name="c", subcore_axis_name="s")
    cp = pltpu.CompilerParams()
    if "needs_layout_passes" in pltpu.CompilerParams.__dataclass_fields__:
        cp = dataclasses.replace(cp, needs_layout_passes=False)

    @functools.partial(
        pl.kernel,
        compiler_params=cp,
        out_type=jax.ShapeDtypeStruct((NC, N, D), jnp.float32),
        mesh=mesh,
        scratch_types=[
            pltpu.VMEM((3, C), jnp.int32),
            pltpu.VMEM((3, C), jnp.int32),
            pltpu.VMEM((C,), jnp.int32),
            pltpu.VMEM((C, D), jnp.float32),
            pltpu.VMEM((C, D), jnp.float32),
            pltpu.VMEM((C, D), jnp.float32),
            pltpu.VMEM_SHARED((N, D), jnp.float32),
            pltpu.SemaphoreType.DMA,
            pltpu.SemaphoreType.DMA,
            pltpu.SemaphoreType.DMA,
            pltpu.SemaphoreType.DMA,
            pltpu.SemaphoreType.DMA,
        ],
    )
    def spmm_kernel(h_hbm, pk_hbm, out_hbm,
                    ib0, ib1, dst_s, g0, g1, s, acc,
                    i_sem0, i_sem1, g_sem0, g_sem1, s_sem):
        core = lax.axis_index("c")
        sub = lax.axis_index("s")
        wid = sub * NC + core

        ibs = (ib0, ib1)
        isems = (i_sem0, i_sem1)
        gbufs = (g0, g1)
        gsems = (g_sem0, g_sem1)
        zi16 = jnp.zeros((LANES,), jnp.int32)
        two16 = jnp.full((LANES,), 2, jnp.int32)
        zeros16 = jnp.zeros((LANES,), jnp.float32)

        def start_idx(c, b):
            pltpu.async_copy(pk_hbm.at[wid, c], ibs[b], isems[b])

        def wait_idx(c, b):
            pltpu.make_async_copy(pk_hbm.at[wid, c], ibs[b], isems[b]).wait()

        def start_gather(b):
            pltpu.async_copy(h_hbm.at[ibs[b].at[0]], gbufs[b], gsems[b])

        def wait_gather(b):
            pltpu.make_async_copy(
                h_hbm.at[ibs[b].at[0]], gbufs[b], gsems[b]).wait()

        def start_scatter():
            pltpu.async_copy(s, acc.at[dst_s], s_sem, add=True)

        def wait_scatter():
            pltpu.make_async_copy(s, acc.at[dst_s], s_sem).wait()

        def copy_dst(b):
            for q in range(C // LANES):
                sl = pl.ds(q * LANES, LANES)
                dst_s[sl] = ibs[b][1, sl]

        iota16 = lax.iota(jnp.int32, LANES)

        def scale(c, b):
            gb = gbufs[b]

            @plsc.parallel_loop(0, C, LANES)
            def _scale(e0):
                wbits = plsc.load_gather(ibs[b], [two16, iota16 + e0])
                wf = plsc.bitcast(wbits, jnp.float32)
                dn = lax.GatherDimensionNumbers(
                    offset_dims=(), collapsed_slice_dims=(0,),
                    start_index_map=(0,))
                for k in range(LANES):
                    kidx = jnp.full((LANES, 1), k, jnp.int32)
                    wsplat = lax.gather(
                        wf, kidx, dn, slice_sizes=(1,),
                        mode=lax.GatherScatterMode.PROMISE_IN_BOUNDS)
                    e = e0 + k
                    for j in range(D // LANES):
                        sl = (e, pl.ds(j * LANES, LANES))
                        s[sl] = gb[sl] * wsplat

        start_idx(0, 0)
        start_idx(1, 1)

        @pl.loop(0, RB)
        def _zero_fill(r):
            for j in range(D // LANES):
                g0[r, pl.ds(j * LANES, LANES)] = zeros16

        @pl.loop(sub, NRB, step=NS)
        def _zero_acc(b):
            pltpu.sync_copy(g0, acc.at[pl.ds(b * RB, RB)])

        plsc.subcore_barrier()

        wait_idx(0, 0)
        start_gather(0)

        @pl.loop(0, NCHUNK - 1, step=2)
        def _edge_chunks(ci):
            for b in range(2):
                c = ci + b
                wait_idx(c + 1, 1 - b)
                start_gather(1 - b)

                wait_gather(b)

                @pl.when(c > 0)
                def _():
                    wait_scatter()

                copy_dst(b)
                scale(c, b)
                start_scatter()

                if b == 0:
                    start_idx(c + 2, b)
                else:
                    @pl.when(c + 2 < NCHUNK)
                    def _():
                        start_idx(c + 2, b)

        wait_gather(0)
        wait_scatter()
        copy_dst(0)
        scale(NCHUNK - 1, 0)
        start_scatter()
        wait_scatter()

        plsc.subcore_barrier()

        @pl.loop(sub, NRB, step=NS)
        def _write_out(b):
            pltpu.sync_copy(acc.at[pl.ds(b * RB, RB)],
                            out_hbm.at[core, pl.ds(b * RB, RB)])

    return spmm_kernel(h, pk)


def _matmul(x, wt):
    rows = 1000
    return pl.pallas_call(
        _matmul_body,
        out_shape=jax.ShapeDtypeStruct((N, D), jnp.float32),
        grid=(N // rows,),
        in_specs=[
            pl.BlockSpec((rows, D), lambda i: (i, 0)),
            pl.BlockSpec((D, D), lambda i: (0, 0)),
        ],
        out_specs=pl.BlockSpec((rows, D), lambda i: (i, 0)),
    )(x, wt)


def _combine(p0, p1):
    rows = 1000
    return pl.pallas_call(
        _combine_body,
        out_shape=jax.ShapeDtypeStruct((N, D), jnp.float32),
        grid=(N // rows,),
        in_specs=[
            pl.BlockSpec((rows, D), lambda i: (i, 0)),
            pl.BlockSpec((rows, D), lambda i: (i, 0)),
        ],
        out_specs=pl.BlockSpec((rows, D), lambda i: (i, 0)),
    )(p0, p1)


@jax.jit
def kernel(x, edge_index, edge_weight, W):
    h = _matmul(x, W.T)
    src = edge_index[1].reshape(NW, NCHUNK, C)
    dst = edge_index[0].reshape(NW, NCHUNK, C)
    wbits = lax.bitcast_convert_type(edge_weight,
                                     jnp.int32).reshape(NW, NCHUNK, C)
    pk = jnp.stack([src, dst, wbits], axis=2)
    partial = _sc_spmm(h, pk)
    return _combine(partial[0], partial[1])


# speedup vs baseline: 9.1591x; 1.1254x over previous
import dataclasses
import functools

import jax
import jax.numpy as jnp
from jax import lax
from jax.experimental import pallas as pl
from jax.experimental.pallas import tpu as pltpu
from jax.experimental.pallas import tpu_sc as plsc

N = 10000
E = 320000
D = 128

NC = 2
NS = 16
NW = NC * NS
EPW = E // NW
C = 80
NCHUNK = EPW // C
RB = 80
NRB = N // RB
LANES = 16


def _matmul_body(x_ref, wt_ref, h_ref):
    h_ref[...] = jnp.dot(x_ref[...], wt_ref[...],
                         preferred_element_type=jnp.float32)


def _combine_body(p0_ref, p1_ref, o_ref):
    o_ref[...] = p0_ref[...] + p1_ref[...]


def _sc_spmm(h, pk):
    mesh = plsc.VectorSubcoreMesh(core_axis_name="c", subcore_axis_name="s")
    cp = pltpu.CompilerParams()
    if "needs_layout_passes" in pltpu.CompilerParams.__dataclass_fields__:
        cp = dataclasses.replace(cp, needs_layout_passes=False)

    @functools.partial(
        pl.kernel,
        compiler_params=cp,
        out_type=jax.ShapeDtypeStruct((NC, N, D), jnp.float32),
        mesh=mesh,
        scratch_types=[
            pltpu.VMEM((3, C), jnp.int32),
            pltpu.VMEM((3, C), jnp.int32),
            pltpu.VMEM((3, C), jnp.int32),
            pltpu.VMEM((C,), jnp.int32),
            pltpu.VMEM((C,), jnp.int32),
            pltpu.VMEM((C,), jnp.int32),
            pltpu.VMEM((C, D), jnp.float32),
            pltpu.VMEM((C, D), jnp.float32),
            pltpu.VMEM((C, D), jnp.float32),
            pltpu.VMEM_SHARED((N, D), jnp.float32),
            pltpu.SemaphoreType.DMA,
            pltpu.SemaphoreType.DMA,
            pltpu.SemaphoreType.DMA,
            pltpu.SemaphoreType.DMA,
            pltpu.SemaphoreType.DMA,
            pltpu.SemaphoreType.DMA,
            pltpu.SemaphoreType.DMA,
            pltpu.SemaphoreType.DMA,
            pltpu.SemaphoreType.DMA,
        ],
    )
    def spmm_kernel(h_hbm, pk_hbm, out_hbm,
                    ib0, ib1, ib2, ds0, ds1, ds2, g0, g1, g2, acc,
                    i_sem0, i_sem1, i_sem2, g_sem0, g_sem1, g_sem2,
                    s_sem0, s_sem1, s_sem2):
        core = lax.axis_index("c")
        sub = lax.axis_index("s")
        wid = sub * NC + core

        ibs = (ib0, ib1, ib2)
        dsts = (ds0, ds1, ds2)
        isems = (i_sem0, i_sem1, i_sem2)
        gbufs = (g0, g1, g2)
        gsems = (g_sem0, g_sem1, g_sem2)
        ssems = (s_sem0, s_sem1, s_sem2)
        two16 = jnp.full((LANES,), 2, jnp.int32)
        zeros16 = jnp.zeros((LANES,), jnp.float32)
        iota16 = lax.iota(jnp.int32, LANES)

        def start_idx(c, b):
            pltpu.async_copy(pk_hbm.at[wid, c], ibs[b], isems[b])

        def wait_idx(c, b):
            pltpu.make_async_copy(pk_hbm.at[wid, c], ibs[b], isems[b]).wait()

        def start_gather(b):
            pltpu.async_copy(h_hbm.at[ibs[b].at[0]], gbufs[b], gsems[b])

        def wait_gather(b):
            pltpu.make_async_copy(
                h_hbm.at[ibs[b].at[0]], gbufs[b], gsems[b]).wait()

        def start_scatter(b):
            pltpu.async_copy(gbufs[b], acc.at[dsts[b]], ssems[b], add=True)

        def wait_scatter(b):
            pltpu.make_async_copy(gbufs[b], acc.at[dsts[b]], ssems[b]).wait()

        def copy_dst(b):
            for q in range(C // LANES):
                sl = pl.ds(q * LANES, LANES)
                dsts[b][sl] = ibs[b][1, sl]

        def scale(c, b):
            gb = gbufs[b]

            @plsc.parallel_loop(0, C, LANES)
            def _scale(e0):
                wbits = plsc.load_gather(ibs[b], [two16, iota16 + e0])
                wf = plsc.bitcast(wbits, jnp.float32)
                dn = lax.GatherDimensionNumbers(
                    offset_dims=(), collapsed_slice_dims=(0,),
                    start_index_map=(0,))
                for k in range(LANES):
                    kidx = jnp.full((LANES, 1), k, jnp.int32)
                    wsplat = lax.gather(
                        wf, kidx, dn, slice_sizes=(1,),
                        mode=lax.GatherScatterMode.PROMISE_IN_BOUNDS)
                    e = e0 + k
                    for j in range(D // LANES):
                        sl = (e, pl.ds(j * LANES, LANES))
                        gb[sl] = gb[sl] * wsplat

        start_idx(0, 0)
        start_idx(1, 1)
        start_idx(2, 2)

        @pl.loop(0, RB)
        def _zero_fill(r):
            for j in range(D // LANES):
                g0[r, pl.ds(j * LANES, LANES)] = zeros16

        @pl.loop(sub, NRB, step=NS)
        def _zero_acc(b):
            pltpu.sync_copy(g0, acc.at[pl.ds(b * RB, RB)])

        plsc.subcore_barrier()

        wait_idx(0, 0)
        start_gather(0)

        def chunk_body(c, b, bn, with_next_idx):
            wait_idx(c + 1, bn)

            @pl.when(c >= 2)
            def _():
                wait_scatter(bn)

            start_gather(bn)

            wait_gather(b)
            copy_dst(b)
            scale(c, b)
            start_scatter(b)
            if with_next_idx:
                start_idx(c + 3, b)

        @pl.loop(0, NCHUNK - 2, step=3)
        def _edge_chunks(ci):
            for b in range(3):
                c = ci + b
                if b == 2:
                    chunk_body(c, b, (b + 1) % 3, False)

                    @pl.when(c + 3 < NCHUNK)
                    def _():
                        start_idx(c + 3, b)
                else:
                    chunk_body(c, b, (b + 1) % 3, True)

        cA = NCHUNK - 2
        wait_idx(cA + 1, 1)
        wait_scatter(1)
        start_gather(1)
        wait_gather(0)
        copy_dst(0)
        scale(cA, 0)
        start_scatter(0)

        cB = NCHUNK - 1
        wait_gather(1)
        copy_dst(1)
        scale(cB, 1)
        start_scatter(1)

        wait_scatter(2)
        wait_scatter(0)
        wait_scatter(1)

        plsc.subcore_barrier()

        @pl.loop(sub, NRB, step=NS)
        def _write_out(b):
            pltpu.sync_copy(acc.at[pl.ds(b * RB, RB)],
                            out_hbm.at[core, pl.ds(b * RB, RB)])

    return spmm_kernel(h, pk)


def _matmul(x, wt):
    rows = 1000
    return pl.pallas_call(
        _matmul_body,
        out_shape=jax.ShapeDtypeStruct((N, D), jnp.float32),
        grid=(N // rows,),
        in_specs=[
            pl.BlockSpec((rows, D), lambda i: (i, 0)),
            pl.BlockSpec((D, D), lambda i: (0, 0)),
        ],
        out_specs=pl.BlockSpec((rows, D), lambda i: (i, 0)),
    )(x, wt)


def _combine(p0, p1):
    rows = 1000
    return pl.pallas_call(
        _combine_body,
        out_shape=jax.ShapeDtypeStruct((N, D), jnp.float32),
        grid=(N // rows,),
        in_specs=[
            pl.BlockSpec((rows, D), lambda i: (i, 0)),
            pl.BlockSpec((rows, D), lambda i: (i, 0)),
        ],
        out_specs=pl.BlockSpec((rows, D), lambda i: (i, 0)),
    )(p0, p1)


@jax.jit
def kernel(x, edge_index, edge_weight, W):
    h = _matmul(x, W.T)
    src = edge_index[1].reshape(NW, NCHUNK, C)
    dst = edge_index[0].reshape(NW, NCHUNK, C)
    wbits = lax.bitcast_convert_type(edge_weight,
                                     jnp.int32).reshape(NW, NCHUNK, C)
    pk = jnp.stack([src, dst, wbits], axis=2)
    partial = _sc_spmm(h, pk)
    return _combine(partial[0], partial[1])
